# baseline (speedup 1.0000x reference)
import functools

import jax
import jax.numpy as jnp
from jax import lax
from jax.experimental import pallas as pl
from jax.experimental.pallas import tpu as pltpu
from jax.experimental.pallas import tpu_sc as plsc

_NC = 2
_NS = 16
_LANES = 16


def _sc_mesh():
    return plsc.VectorSubcoreMesh(core_axis_name="c", subcore_axis_name="s")


def _pad_rows(n):
    return -(-(n + 1) // (_NS * 8)) * (_NS * 8)


def _make_deg_kernel(n2, nb):
    chunk = n2 // _NS

    @functools.partial(
        pl.kernel,
        mesh=_sc_mesh(),
        out_type=jax.ShapeDtypeStruct((_NC, n2, 128), jnp.float32),
        scratch_types=[
            pltpu.VMEM((nb, 128), jnp.int32),
            pltpu.VMEM((128, 128), jnp.float32),
            pltpu.VMEM_SHARED((n2, 128), jnp.float32),
        ],
    )
    def deg_kernel(col_hbm, ones_hbm, zeros_hbm, out_hbm, col_v, ones_v, acc):
        c = lax.axis_index("c")
        s = lax.axis_index("s")
        w = c * _NS + s
        pltpu.sync_copy(col_hbm.at[w], col_v)
        pltpu.sync_copy(ones_hbm, ones_v)
        pltpu.sync_copy(zeros_hbm, acc.at[pl.ds(s * chunk, chunk)])
        plsc.subcore_barrier()

        def batch(b, carry):
            pltpu.sync_copy(ones_v, acc.at[col_v.at[b]], add=True)
            return carry

        lax.fori_loop(0, nb, batch, 0)
        plsc.subcore_barrier()
        pltpu.sync_copy(acc.at[pl.ds(s * chunk, chunk)],
                        out_hbm.at[c, pl.ds(s * chunk, chunk)])

    return deg_kernel


def _make_spmm_kernel(n2, nb, src_rows):
    chunk = n2 // _NS
    d = 128

    @functools.partial(
        pl.kernel,
        mesh=_sc_mesh(),
        out_type=jax.ShapeDtypeStruct((_NC, n2, d), jnp.float32),
        scratch_types=[
            pltpu.VMEM((nb, 128), jnp.int32),
            pltpu.VMEM((1, 128), jnp.int32),
            pltpu.VMEM((1, 128), jnp.int32),
            pltpu.VMEM((1, 128), jnp.float32),
            pltpu.VMEM((1, 128), jnp.float32),
            pltpu.VMEM((128, d), jnp.float32),
            pltpu.VMEM((128, d), jnp.float32),
            pltpu.VMEM_SHARED((n2, d), jnp.float32),
            pltpu.SemaphoreType.DMA,
            pltpu.SemaphoreType.DMA,
            pltpu.SemaphoreType.DMA,
            pltpu.SemaphoreType.DMA,
            pltpu.SemaphoreType.DMA,
            pltpu.SemaphoreType.DMA,
        ],
    )
    def spmm_kernel(src_hbm, row_hbm, col_hbm, cc_hbm, zeros_hbm, out_hbm,
                    row_v, col_a, col_b, cc_a, cc_b, rows_a, rows_b, acc,
                    sem_a, sem_b, sem_ca, sem_cb, sem_da, sem_db):
        c = lax.axis_index("c")
        s = lax.axis_index("s")
        w = c * _NS + s
        pltpu.sync_copy(row_hbm.at[w], row_v)
        pltpu.sync_copy(zeros_hbm, acc.at[pl.ds(s * chunk, chunk)])
        plsc.subcore_barrier()

        def fetch(b, rows_buf, cc_buf, col_buf, sem_r, sem_c, sem_d):
            pltpu.async_copy(src_hbm.at[row_v.at[b]], rows_buf, sem_r)
            pltpu.async_copy(cc_hbm.at[w, b], cc_buf, sem_c)
            pltpu.async_copy(col_hbm.at[w, b], col_buf, sem_d)

        def process(b, rows_buf, cc_buf, col_buf, sem_r, sem_c, sem_d):
            pltpu.make_async_copy(src_hbm.at[row_v.at[b]], rows_buf,
                                  sem_r).wait()
            pltpu.make_async_copy(cc_hbm.at[w, b], cc_buf, sem_c).wait()
            pltpu.make_async_copy(col_hbm.at[w, b], col_buf, sem_d).wait()

            def group(g, gcarry):
                cvec = cc_buf[0, pl.ds(g * _LANES, _LANES)]
                for jj in range(_LANES):
                    cf = jnp.full((_LANES,), cvec[jj], dtype=jnp.float32)
                    j = g * _LANES + jj
                    for kk in range(d // _LANES):
                        sl = pl.ds(kk * _LANES, _LANES)
                        rows_buf[j, sl] = rows_buf[j, sl] * cf
                return gcarry

            lax.fori_loop(0, 128 // _LANES, group, 0)
            pltpu.sync_copy(rows_buf, acc.at[col_buf.at[0]], add=True)

        fetch(0, rows_a, cc_a, col_a, sem_a, sem_ca, sem_da)

        def pair(i, carry):
            b0 = 2 * i
            fetch(b0 + 1, rows_b, cc_b, col_b, sem_b, sem_cb, sem_db)
            process(b0, rows_a, cc_a, col_a, sem_a, sem_ca, sem_da)
            fetch(jnp.minimum(b0 + 2, nb - 1), rows_a, cc_a, col_a,
                  sem_a, sem_ca, sem_da)
            process(b0 + 1, rows_b, cc_b, col_b, sem_b, sem_cb, sem_db)
            return carry

        lax.fori_loop(0, nb // 2, pair, 0)
        if nb % 2 == 1:
            process(nb - 1, rows_a, cc_a, col_a, sem_a, sem_ca, sem_da)
        else:
            pltpu.make_async_copy(src_hbm.at[row_v.at[nb - 1]], rows_a,
                                  sem_a).wait()
            pltpu.make_async_copy(cc_hbm.at[w, nb - 1], cc_a, sem_ca).wait()
            pltpu.make_async_copy(col_hbm.at[w, nb - 1], col_a, sem_da).wait()
        plsc.subcore_barrier()
        pltpu.sync_copy(acc.at[pl.ds(s * chunk, chunk)],
                        out_hbm.at[c, pl.ds(s * chunk, chunk)])

    return spmm_kernel


def _tca_body(lmda_ref, deg_ref, x_ref, w1_ref, ew_ref,
              hs_ref, dinv_ref, cc_ref):
    lam = lmda_ref[0, 0]
    deg = deg_ref[0, :, 0] + deg_ref[1, :, 0]
    dv = jnp.where(deg > 0.0, lax.rsqrt(jnp.maximum(deg, 1e-12)), 0.0)
    dinv_ref[:, 0] = dv
    h = jnp.dot(x_ref[:, :], w1_ref[:, :], preferred_element_type=jnp.float32)
    hs_ref[0] = h * dv[:, None]
    cc_ref[...] = lam + (1.0 - lam) * ew_ref[...]


def _tcb_body(agg_ref, dinv_ref, b1_ref, w2_ref, h2s_ref):
    dv = dinv_ref[:, 0]
    t = jnp.concatenate([agg_ref[0], agg_ref[1]], axis=1)
    t = t * dv[:, None] + b1_ref[0, :][None, :]
    t = jnp.maximum(t, 0.0)
    h2 = jnp.dot(t, w2_ref[:, :], preferred_element_type=jnp.float32)
    h2s_ref[:, :] = h2 * dv[:, None]


def _tcc_body(p_ref, dinv_ref, b2_ref, out_ref):
    dv = dinv_ref[:, 0]
    out_ref[:, :] = ((p_ref[0] + p_ref[1]) * dv[:, None]
                     + b2_ref[0, :][None, :])


def kernel(x, edge_index, edge_weight, lmda, W1, b1, W2, b2):
    n, d_in = x.shape
    e = edge_index.shape[1]
    d_hid = W1.shape[1]
    d_out = W2.shape[1]
    n2 = _pad_rows(n)

    row = edge_index[0]
    col = edge_index[1]

    def _pack(arr, nt, pad_val):
        per = e // nt
        nb = -(-per // 128)
        pad = nb * 128 - per
        a2 = arr.reshape(nt, per)
        padv = jnp.full((nt, pad), pad_val, arr.dtype)
        return jnp.concatenate([a2, padv], axis=1).reshape(nt, nb, 128), nb

    col_deg, nbdeg = _pack(col, _NC * _NS, jnp.int32(n))
    ones_rows = jnp.ones((128, 128), jnp.float32)
    zeros128 = jnp.zeros((n2 // _NS, 128), jnp.float32)
    deg_part = _make_deg_kernel(n2, nbdeg)(col_deg, ones_rows, zeros128)

    nb_rows = 10
    br = n // nb_rows
    half = d_hid // 2
    bre = e // (128 * nb_rows)
    ew2 = edge_weight.reshape(nb_rows, bre, 128)
    lmda2 = jnp.reshape(lmda, (1, 1))
    hs, dinv, cc3 = pl.pallas_call(
        _tca_body,
        grid=(nb_rows, 2),
        in_specs=[
            pl.BlockSpec(memory_space=pltpu.SMEM),
            pl.BlockSpec((_NC, br, 128), lambda i, j: (0, i, 0)),
            pl.BlockSpec((br, d_in), lambda i, j: (i, 0)),
            pl.BlockSpec((d_in, half), lambda i, j: (0, j)),
            pl.BlockSpec((1, bre, 128), lambda i, j: (i, 0, 0)),
        ],
        out_specs=[
            pl.BlockSpec((1, br, half), lambda i, j: (j, i, 0)),
            pl.BlockSpec((br, 1), lambda i, j: (i, 0)),
            pl.BlockSpec((1, bre, 128), lambda i, j: (i, 0, 0)),
        ],
        out_shape=[
            jax.ShapeDtypeStruct((2, n, half), jnp.float32),
            jax.ShapeDtypeStruct((n, 1), jnp.float32),
            jax.ShapeDtypeStruct((nb_rows, bre, 128), jnp.float32),
        ],
    )(lmda2, deg_part, x, W1, ew2)
    cc = cc3.reshape(e)
    hs2 = hs.reshape(2 * n, half)

    rowt, nb1 = _pack(row, _NS, jnp.int32(0))
    row1 = jnp.concatenate([rowt, rowt + jnp.int32(n)], axis=0)
    colt, _ = _pack(col, _NS, jnp.int32(0))
    col1 = jnp.concatenate([colt, colt], axis=0).reshape(
        _NC * _NS, nb1, 1, 128)
    cct, _ = _pack(cc, _NS, jnp.float32(0))
    cc1 = jnp.concatenate([cct, cct], axis=0).reshape(_NC * _NS, nb1, 1, 128)
    agg = _make_spmm_kernel(n2, nb1, 2 * n)(hs2, row1, col1, cc1, zeros128)

    b1_2 = b1.reshape(1, d_hid)
    h2s = pl.pallas_call(
        _tcb_body,
        grid=(nb_rows,),
        in_specs=[
            pl.BlockSpec((_NC, br, half), lambda i: (0, i, 0)),
            pl.BlockSpec((br, 1), lambda i: (i, 0)),
            pl.BlockSpec((1, d_hid), lambda i: (0, 0)),
            pl.BlockSpec((d_hid, d_out), lambda i: (0, 0)),
        ],
        out_specs=pl.BlockSpec((br, d_out), lambda i: (i, 0)),
        out_shape=jax.ShapeDtypeStruct((n, d_out), jnp.float32),
    )(agg, dinv, b1_2, W2)

    row2, nb2 = _pack(row, _NC * _NS, jnp.int32(0))
    col2, _ = _pack(col, _NC * _NS, jnp.int32(0))
    col2 = col2.reshape(_NC * _NS, nb2, 1, 128)
    cc2, _ = _pack(cc, _NC * _NS, jnp.float32(0))
    cc2 = cc2.reshape(_NC * _NS, nb2, 1, 128)
    part = _make_spmm_kernel(n2, nb2, n)(h2s, row2, col2, cc2, zeros128)

    b2_2 = b2.reshape(1, d_out)
    out = pl.pallas_call(
        _tcc_body,
        grid=(nb_rows,),
        in_specs=[
            pl.BlockSpec((_NC, br, d_out), lambda i: (0, i, 0)),
            pl.BlockSpec((br, 1), lambda i: (i, 0)),
            pl.BlockSpec((1, d_out), lambda i: (0, 0)),
        ],
        out_specs=pl.BlockSpec((br, d_out), lambda i: (i, 0)),
        out_shape=jax.ShapeDtypeStruct((n, d_out), jnp.float32),
    )(part, dinv, b2_2)
    return out


# speedup vs baseline: 1.0187x; 1.0187x over previous
import functools

import jax
import jax.numpy as jnp
from jax import lax
from jax.experimental import pallas as pl
from jax.experimental.pallas import tpu as pltpu
from jax.experimental.pallas import tpu_sc as plsc

_NC = 2
_NS = 16
_LANES = 16


def _sc_mesh():
    return plsc.VectorSubcoreMesh(core_axis_name="c", subcore_axis_name="s")


def _pad_rows(n):
    return -(-(n + 1) // (_NS * 8)) * (_NS * 8)


def _make_deg_kernel(n2, nb):
    chunk = n2 // _NS

    @functools.partial(
        pl.kernel,
        mesh=_sc_mesh(),
        out_type=jax.ShapeDtypeStruct((_NC, n2, 128), jnp.float32),
        scratch_types=[
            pltpu.VMEM((nb, 128), jnp.int32),
            pltpu.VMEM((128, 128), jnp.float32),
            pltpu.VMEM_SHARED((n2, 128), jnp.float32),
        ],
    )
    def deg_kernel(col_hbm, ones_hbm, zeros_hbm, out_hbm, col_v, ones_v, acc):
        c = lax.axis_index("c")
        s = lax.axis_index("s")
        w = c * _NS + s
        pltpu.sync_copy(col_hbm.at[w], col_v)
        pltpu.sync_copy(ones_hbm, ones_v)
        pltpu.sync_copy(zeros_hbm, acc.at[pl.ds(s * chunk, chunk)])
        plsc.subcore_barrier()

        def batch(b, carry):
            pltpu.sync_copy(ones_v, acc.at[col_v.at[b]], add=True)
            return carry

        lax.fori_loop(0, nb, batch, 0)
        plsc.subcore_barrier()
        pltpu.sync_copy(acc.at[pl.ds(s * chunk, chunk)],
                        out_hbm.at[c, pl.ds(s * chunk, chunk)])

    return deg_kernel


def _make_spmm_kernel(n2, nb, src_rows):
    chunk = n2 // _NS
    d = 128

    @functools.partial(
        pl.kernel,
        mesh=_sc_mesh(),
        out_type=jax.ShapeDtypeStruct((_NC, n2, d), jnp.float32),
        scratch_types=[
            pltpu.VMEM((nb, 128), jnp.int32),
            pltpu.VMEM((1, 128), jnp.int32),
            pltpu.VMEM((1, 128), jnp.int32),
            pltpu.VMEM((1, 128), jnp.float32),
            pltpu.VMEM((1, 128), jnp.float32),
            pltpu.VMEM((128, d), jnp.float32),
            pltpu.VMEM((128, d), jnp.float32),
            pltpu.VMEM_SHARED((n2, d), jnp.float32),
            pltpu.SemaphoreType.DMA,
            pltpu.SemaphoreType.DMA,
            pltpu.SemaphoreType.DMA,
            pltpu.SemaphoreType.DMA,
            pltpu.SemaphoreType.DMA,
            pltpu.SemaphoreType.DMA,
        ],
    )
    def spmm_kernel(src_hbm, row_hbm, col_hbm, cc_hbm, zeros_hbm, out_hbm,
                    row_v, col_a, col_b, cc_a, cc_b, rows_a, rows_b, acc,
                    sem_a, sem_b, sem_ca, sem_cb, sem_da, sem_db):
        c = lax.axis_index("c")
        s = lax.axis_index("s")
        w = c * _NS + s
        pltpu.sync_copy(row_hbm.at[w], row_v)
        pltpu.sync_copy(zeros_hbm, acc.at[pl.ds(s * chunk, chunk)])
        plsc.subcore_barrier()

        def fetch(b, rows_buf, cc_buf, col_buf, sem_r, sem_c, sem_d):
            pltpu.async_copy(src_hbm.at[row_v.at[b]], rows_buf, sem_r)
            pltpu.async_copy(cc_hbm.at[w, b], cc_buf, sem_c)
            pltpu.async_copy(col_hbm.at[w, b], col_buf, sem_d)

        def process(b, rows_buf, cc_buf, col_buf, sem_r, sem_c, sem_d):
            pltpu.make_async_copy(src_hbm.at[row_v.at[b]], rows_buf,
                                  sem_r).wait()
            pltpu.make_async_copy(cc_hbm.at[w, b], cc_buf, sem_c).wait()
            pltpu.make_async_copy(col_hbm.at[w, b], col_buf, sem_d).wait()

            def group(g, gcarry):
                cvec = cc_buf[0, pl.ds(g * _LANES, _LANES)]
                for jj in range(_LANES):
                    cf = jnp.full((_LANES,), cvec[jj], dtype=jnp.float32)
                    j = g * _LANES + jj
                    for kk in range(d // _LANES):
                        sl = pl.ds(kk * _LANES, _LANES)
                        rows_buf[j, sl] = rows_buf[j, sl] * cf
                return gcarry

            lax.fori_loop(0, 128 // _LANES, group, 0)
            pltpu.sync_copy(rows_buf, acc.at[col_buf.at[0]], add=True)

        fetch(0, rows_a, cc_a, col_a, sem_a, sem_ca, sem_da)

        def pair(i, carry):
            b0 = 2 * i
            fetch(b0 + 1, rows_b, cc_b, col_b, sem_b, sem_cb, sem_db)
            process(b0, rows_a, cc_a, col_a, sem_a, sem_ca, sem_da)
            fetch(jnp.minimum(b0 + 2, nb - 1), rows_a, cc_a, col_a,
                  sem_a, sem_ca, sem_da)
            process(b0 + 1, rows_b, cc_b, col_b, sem_b, sem_cb, sem_db)
            return carry

        lax.fori_loop(0, nb // 2, pair, 0)
        if nb % 2 == 1:
            process(nb - 1, rows_a, cc_a, col_a, sem_a, sem_ca, sem_da)
        else:
            pltpu.make_async_copy(src_hbm.at[row_v.at[nb - 1]], rows_a,
                                  sem_a).wait()
            pltpu.make_async_copy(cc_hbm.at[w, nb - 1], cc_a, sem_ca).wait()
            pltpu.make_async_copy(col_hbm.at[w, nb - 1], col_a, sem_da).wait()
        plsc.subcore_barrier()
        pltpu.sync_copy(acc.at[pl.ds(s * chunk, chunk)],
                        out_hbm.at[c, pl.ds(s * chunk, chunk)])

    return spmm_kernel


def _tca1_body(lmda_ref, x_ref, w1_ref, ew_ref, h_ref, cc_ref):
    lam = lmda_ref[0, 0]
    h = jnp.dot(x_ref[:, :], w1_ref[:, :], preferred_element_type=jnp.float32)
    h_ref[0] = h
    cc_ref[...] = lam + (1.0 - lam) * ew_ref[...]


def _tca2_body(deg_ref, h_ref, hs_ref, dinv_ref):
    deg = deg_ref[0, :, 0] + deg_ref[1, :, 0]
    dv = jnp.where(deg > 0.0, lax.rsqrt(jnp.maximum(deg, 1e-12)), 0.0)
    dinv_ref[:, 0] = dv
    hs_ref[...] = h_ref[...] * dv[None, :, None]


def _tcb_body(agg_ref, dinv_ref, b1_ref, w2_ref, h2s_ref):
    dv = dinv_ref[:, 0]
    t = jnp.concatenate([agg_ref[0], agg_ref[1]], axis=1)
    t = t * dv[:, None] + b1_ref[0, :][None, :]
    t = jnp.maximum(t, 0.0)
    h2 = jnp.dot(t, w2_ref[:, :], preferred_element_type=jnp.float32)
    h2s_ref[:, :] = h2 * dv[:, None]


def _tcc_body(p_ref, dinv_ref, b2_ref, out_ref):
    dv = dinv_ref[:, 0]
    out_ref[:, :] = ((p_ref[0] + p_ref[1]) * dv[:, None]
                     + b2_ref[0, :][None, :])


def kernel(x, edge_index, edge_weight, lmda, W1, b1, W2, b2):
    n, d_in = x.shape
    e = edge_index.shape[1]
    d_hid = W1.shape[1]
    d_out = W2.shape[1]
    n2 = _pad_rows(n)

    row = edge_index[0]
    col = edge_index[1]

    def _pack(arr, nt, pad_val):
        per = e // nt
        nb = -(-per // 128)
        pad = nb * 128 - per
        a2 = arr.reshape(nt, per)
        padv = jnp.full((nt, pad), pad_val, arr.dtype)
        return jnp.concatenate([a2, padv], axis=1).reshape(nt, nb, 128), nb

    col_deg, nbdeg = _pack(col, _NC * _NS, jnp.int32(n))
    ones_rows = jnp.ones((128, 128), jnp.float32)
    zeros128 = jnp.zeros((n2 // _NS, 128), jnp.float32)
    deg_part = _make_deg_kernel(n2, nbdeg)(col_deg, ones_rows, zeros128)

    nb_rows = 10
    br = n // nb_rows
    half = d_hid // 2
    bre = e // (128 * nb_rows)
    ew2 = edge_weight.reshape(nb_rows, bre, 128)
    lmda2 = jnp.reshape(lmda, (1, 1))
    h, cc3 = pl.pallas_call(
        _tca1_body,
        grid=(nb_rows, 2),
        in_specs=[
            pl.BlockSpec(memory_space=pltpu.SMEM),
            pl.BlockSpec((br, d_in), lambda i, j: (i, 0)),
            pl.BlockSpec((d_in, half), lambda i, j: (0, j)),
            pl.BlockSpec((1, bre, 128), lambda i, j: (i, 0, 0)),
        ],
        out_specs=[
            pl.BlockSpec((1, br, half), lambda i, j: (j, i, 0)),
            pl.BlockSpec((1, bre, 128), lambda i, j: (i, 0, 0)),
        ],
        out_shape=[
            jax.ShapeDtypeStruct((2, n, half), jnp.float32),
            jax.ShapeDtypeStruct((nb_rows, bre, 128), jnp.float32),
        ],
    )(lmda2, x, W1, ew2)
    hs, dinv = pl.pallas_call(
        _tca2_body,
        grid=(nb_rows,),
        in_specs=[
            pl.BlockSpec((_NC, br, 128), lambda i: (0, i, 0)),
            pl.BlockSpec((_NC, br, half), lambda i: (0, i, 0)),
        ],
        out_specs=[
            pl.BlockSpec((_NC, br, half), lambda i: (0, i, 0)),
            pl.BlockSpec((br, 1), lambda i: (i, 0)),
        ],
        out_shape=[
            jax.ShapeDtypeStruct((2, n, half), jnp.float32),
            jax.ShapeDtypeStruct((n, 1), jnp.float32),
        ],
    )(deg_part, h)
    cc = cc3.reshape(e)
    hs2 = hs.reshape(2 * n, half)

    rowt, nb1 = _pack(row, _NS, jnp.int32(0))
    row1 = jnp.concatenate([rowt, rowt + jnp.int32(n)], axis=0)
    colt, _ = _pack(col, _NS, jnp.int32(0))
    col1 = jnp.concatenate([colt, colt], axis=0).reshape(
        _NC * _NS, nb1, 1, 128)
    cct, _ = _pack(cc, _NS, jnp.float32(0))
    cc1 = jnp.concatenate([cct, cct], axis=0).reshape(_NC * _NS, nb1, 1, 128)
    agg = _make_spmm_kernel(n2, nb1, 2 * n)(hs2, row1, col1, cc1, zeros128)

    b1_2 = b1.reshape(1, d_hid)
    h2s = pl.pallas_call(
        _tcb_body,
        grid=(nb_rows,),
        in_specs=[
            pl.BlockSpec((_NC, br, half), lambda i: (0, i, 0)),
            pl.BlockSpec((br, 1), lambda i: (i, 0)),
            pl.BlockSpec((1, d_hid), lambda i: (0, 0)),
            pl.BlockSpec((d_hid, d_out), lambda i: (0, 0)),
        ],
        out_specs=pl.BlockSpec((br, d_out), lambda i: (i, 0)),
        out_shape=jax.ShapeDtypeStruct((n, d_out), jnp.float32),
    )(agg, dinv, b1_2, W2)

    row2, nb2 = _pack(row, _NC * _NS, jnp.int32(0))
    col2, _ = _pack(col, _NC * _NS, jnp.int32(0))
    col2 = col2.reshape(_NC * _NS, nb2, 1, 128)
    cc2, _ = _pack(cc, _NC * _NS, jnp.float32(0))
    cc2 = cc2.reshape(_NC * _NS, nb2, 1, 128)
    part = _make_spmm_kernel(n2, nb2, n)(h2s, row2, col2, cc2, zeros128)

    b2_2 = b2.reshape(1, d_out)
    out = pl.pallas_call(
        _tcc_body,
        grid=(nb_rows,),
        in_specs=[
            pl.BlockSpec((_NC, br, d_out), lambda i: (0, i, 0)),
            pl.BlockSpec((br, 1), lambda i: (i, 0)),
            pl.BlockSpec((1, d_out), lambda i: (0, 0)),
        ],
        out_specs=pl.BlockSpec((br, d_out), lambda i: (i, 0)),
        out_shape=jax.ShapeDtypeStruct((n, d_out), jnp.float32),
    )(part, dinv, b2_2)
    return out
